# baseline (device time: 222615 ns/iter reference)
import jax
import jax.numpy as jnp
from jax import lax
from jax.experimental import pallas as pl
from jax.experimental.pallas import tpu as pltpu

N_DEV = 8
N_EXP = 32
E_PER = 4
CAP = 204
N_TOK = 1024
D_IN = 256
D_OUT = 512

_sem_signal = getattr(pl, "semaphore_signal", None) or pltpu.semaphore_signal
_sem_wait = getattr(pl, "semaphore_wait", None) or pltpu.semaphore_wait
_CompilerParams = getattr(pltpu, "CompilerParams", None) or pltpu.TPUCompilerParams


def _neighbor_barrier(left, right):
    barrier_sem = pltpu.get_barrier_semaphore()
    for nbr in (left, right):
        _sem_signal(
            barrier_sem,
            inc=1,
            device_id=(nbr,),
            device_id_type=pl.DeviceIdType.MESH,
        )
    _sem_wait(barrier_sem, 2)


def _ag_idx(ridx):

    def body(idx_ref, out_ref, comm_ref, send_sems, recv_sems, credit_sem):
        my = lax.axis_index("i")
        left = lax.rem(my + N_DEV - 1, N_DEV)
        right = lax.rem(my + 1, N_DEV)
        _neighbor_barrier(left, right)

        out_ref[pl.ds(my * 8, 8), :] = idx_ref[...]
        comm_ref[0] = idx_ref[...]

        for h in range(N_DEV - 1):
            send_slot = h % 2
            recv_slot = (h + 1) % 2
            rdma = pltpu.make_async_remote_copy(
                src_ref=comm_ref.at[send_slot],
                dst_ref=comm_ref.at[recv_slot],
                send_sem=send_sems.at[send_slot],
                recv_sem=recv_sems.at[recv_slot],
                device_id=(right,),
                device_id_type=pl.DeviceIdType.MESH,
            )
            if h >= 1:
                _sem_wait(credit_sem, 1)
            rdma.start()
            rdma.wait()
            if h <= N_DEV - 3:
                _sem_signal(
                    credit_sem,
                    inc=1,
                    device_id=(left,),
                    device_id_type=pl.DeviceIdType.MESH,
                )
            origin = lax.rem(my - h - 1 + N_DEV, N_DEV)
            out_ref[pl.ds(origin * 8, 8), :] = comm_ref[recv_slot]

    return pl.pallas_call(
        body,
        out_shape=jax.ShapeDtypeStruct((N_DEV * 8, 128), jnp.int32),
        in_specs=[pl.BlockSpec(memory_space=pltpu.VMEM)],
        out_specs=pl.BlockSpec(memory_space=pltpu.VMEM),
        scratch_shapes=[
            pltpu.VMEM((2, 8, 128), jnp.int32),
            pltpu.SemaphoreType.DMA((2,)),
            pltpu.SemaphoreType.DMA((2,)),
            pltpu.SemaphoreType.REGULAR,
        ],
        compiler_params=_CompilerParams(collective_id=0),
    )(ridx)


def _moe(x, m_hop, w):

    def body(x_ref, m_ref, w_ref, out_ref, comm_ref, send_sems, recv_sems, credit_sem):
        my = lax.axis_index("i")
        left = lax.rem(my + N_DEV - 1, N_DEV)
        right = lax.rem(my + 1, N_DEV)
        _neighbor_barrier(left, right)

        comm_ref[0] = w_ref[...]
        xv = x_ref[...]
        mv = m_ref[...]

        for h in range(N_DEV):
            slot = h % 2
            rdma = None
            if h < N_DEV - 1:
                recv_slot = (h + 1) % 2
                rdma = pltpu.make_async_remote_copy(
                    src_ref=comm_ref.at[slot],
                    dst_ref=comm_ref.at[recv_slot],
                    send_sem=send_sems.at[slot],
                    recv_sem=recv_sems.at[recv_slot],
                    device_id=(right,),
                    device_id_type=pl.DeviceIdType.MESH,
                )
                if h >= 1:
                    _sem_wait(credit_sem, 1)
                rdma.start()

            acc = None
            for j in range(E_PER):
                k = h * E_PER + j
                col = mv[:, k : k + 1]
                t = jnp.dot(
                    xv * col,
                    comm_ref[slot, j],
                    preferred_element_type=jnp.float32,
                )
                acc = t if acc is None else acc + t
            if h == 0:
                out_ref[...] = acc
            else:
                out_ref[...] = out_ref[...] + acc

            if rdma is not None:
                rdma.wait()
                if h <= N_DEV - 3:
                    _sem_signal(
                        credit_sem,
                        inc=1,
                        device_id=(left,),
                        device_id_type=pl.DeviceIdType.MESH,
                    )

    return pl.pallas_call(
        body,
        out_shape=jax.ShapeDtypeStruct((N_TOK, D_OUT), jnp.float32),
        in_specs=[
            pl.BlockSpec(memory_space=pltpu.VMEM),
            pl.BlockSpec(memory_space=pltpu.VMEM),
            pl.BlockSpec(memory_space=pltpu.VMEM),
        ],
        out_specs=pl.BlockSpec(memory_space=pltpu.VMEM),
        scratch_shapes=[
            pltpu.VMEM((2, E_PER, D_IN, D_OUT), jnp.float32),
            pltpu.SemaphoreType.DMA((2,)),
            pltpu.SemaphoreType.DMA((2,)),
            pltpu.SemaphoreType.REGULAR,
        ],
        compiler_params=_CompilerParams(collective_id=1),
    )(x, m_hop, w)


def kernel(x, router_W, route_idx, expert_W):
    del router_W
    p = lax.axis_index("i")

    g = _ag_idx(route_idx.reshape(8, 128))
    gflat = g.reshape(N_DEV * N_TOK)

    eids = jnp.arange(N_EXP, dtype=jnp.int32)
    oh = (gflat[:, None] == eids[None, :]).astype(jnp.int32)
    counts = oh.reshape(N_DEV, N_TOK, N_EXP).sum(axis=1)
    base = jnp.cumsum(counts, axis=0) - counts
    mybase = lax.dynamic_slice(base, (p, 0), (1, N_EXP))

    lo = route_idx == eids[None, :]
    loi = lo.astype(jnp.int32)
    lrank = jnp.cumsum(loi, axis=0) - loi
    rank = mybase + lrank
    mask = (lo & (rank < CAP)).astype(jnp.float32)

    shard_order = jnp.remainder(p - jnp.arange(N_DEV, dtype=jnp.int32), N_DEV)
    order = (
        shard_order[:, None] * E_PER + jnp.arange(E_PER, dtype=jnp.int32)[None, :]
    ).reshape(N_EXP)
    m_hop = jnp.take(mask, order, axis=1)

    return _moe(x, m_hop, expert_W)
